# baseline (speedup 1.0000x reference)
import jax
import jax.numpy as jnp
from jax.experimental import pallas as pl
from jax.experimental.pallas import tpu as pltpu

_N = 10000
_BLK = 1000


def _lstm_kernel(x_ref, h_ref, c_ref, wx_ref, wh_ref, b_ref,
                 wci_ref, wcf_ref, wco_ref, fcw_ref, fcb_ref,
                 out_ref, hn_ref, cn_ref):
    x = x_ref[...]
    h = h_ref[...]
    c = c_ref[...]
    pre = jnp.dot(x, wx_ref[...], preferred_element_type=jnp.float32)
    pre = pre + jnp.dot(h, wh_ref[...], preferred_element_type=jnp.float32)
    pre = pre + b_ref[...]
    i_g = jax.nn.sigmoid(pre[:, 0:32] + wci_ref[...] * c)
    f_g = jax.nn.sigmoid(pre[:, 32:64] + wcf_ref[...] * c)
    t_g = jnp.tanh(pre[:, 64:96])
    c_new = f_g * c + i_g * t_g
    o_g = jax.nn.sigmoid(pre[:, 96:128] + wco_ref[...] * c_new)
    h_new = o_g * jnp.tanh(c_new)
    cn_ref[...] = c_new
    hn_ref[...] = h_new
    relu_h = jnp.maximum(h_new, 0.0)
    out_ref[...] = (jnp.sum(relu_h * fcw_ref[...], axis=1, keepdims=True)
                    + fcb_ref[...])


def kernel(x, edge_index, edge_weight, h, c,
           W_xi, b_xi, W_hi, b_hi, W_xf, b_xf, W_hf, b_hf,
           W_xc, b_xc, W_hc, b_hc, W_xo, b_xo, W_ho, b_ho,
           w_ci, w_cf, w_co, b_i, b_f, b_c, b_o, fc_w, fc_b):
    del edge_index, edge_weight
    f_in = x.shape[1]
    h_dim = h.shape[1]
    wx = jnp.concatenate([W_xi, W_xf, W_xc, W_xo], axis=1)
    wh = jnp.concatenate([W_hi, W_hf, W_hc, W_ho], axis=1)
    bias = jnp.concatenate([b_xi + b_hi + b_i[0],
                            b_xf + b_hf + b_f[0],
                            b_xc + b_hc + b_c[0],
                            b_xo + b_ho + b_o[0]])[None, :]
    fcw = fc_w.T
    fcb = fc_b.reshape(1, 1)

    n = x.shape[0]
    grid = (n // _BLK,)
    row = lambda i: (i, 0)
    full = lambda i: (0, 0)
    out, h_new, c_new = pl.pallas_call(
        _lstm_kernel,
        grid=grid,
        in_specs=[
            pl.BlockSpec((_BLK, f_in), row),
            pl.BlockSpec((_BLK, h_dim), row),
            pl.BlockSpec((_BLK, h_dim), row),
            pl.BlockSpec((f_in, 4 * h_dim), full),
            pl.BlockSpec((h_dim, 4 * h_dim), full),
            pl.BlockSpec((1, 4 * h_dim), full),
            pl.BlockSpec((1, h_dim), full),
            pl.BlockSpec((1, h_dim), full),
            pl.BlockSpec((1, h_dim), full),
            pl.BlockSpec((1, h_dim), full),
            pl.BlockSpec((1, 1), full),
        ],
        out_specs=[
            pl.BlockSpec((_BLK, 1), row),
            pl.BlockSpec((_BLK, h_dim), row),
            pl.BlockSpec((_BLK, h_dim), row),
        ],
        out_shape=[
            jax.ShapeDtypeStruct((n, 1), jnp.float32),
            jax.ShapeDtypeStruct((n, h_dim), jnp.float32),
            jax.ShapeDtypeStruct((n, h_dim), jnp.float32),
        ],
        compiler_params=pltpu.CompilerParams(
            dimension_semantics=("arbitrary",),
        ),
    )(x, h, c, wx, wh, bias, w_ci, w_cf, w_co, fcw, fcb)
    return (out, h_new, c_new)
